# baseline (speedup 1.0000x reference)
import jax
import jax.numpy as jnp
from jax import lax
from jax.experimental import pallas as pl
from jax.experimental.pallas import tpu as pltpu
from jax.experimental.pallas import tpu_sc as plsc

NUM_EMBEDDINGS = 1000000
D = 32
HW = D // 2
PADDING_IDX = 0
B = 16384
HIST = 50

L = 16
NC, NS = 2, 16
NW = NC * NS
RW = B // NW
CB = 32
GC = CB * HIST
NCHUNK = RW // CB

SB = 512
NSB = NUM_EMBEDDINGS // SB
TAIL_H = NUM_EMBEDDINGS - NSB * SB
SB_PER_W = 62


def _tbody(tt_hbm, out_hbm, b0, b1, b2, b3, o0, o1,
           si0, si1, si2, si3, so0, so1, tb, tob):
    wid = lax.axis_index("s") * NC + lax.axis_index("c")
    start = 61 * wid + jnp.minimum(wid, 1)
    cnt = jnp.where(wid < 1, 62, 61)
    bufs = (b0, b1, b2, b3)
    obufs = (o0, o1)
    semi = (si0, si1, si2, si3)
    semo = (so0, so1)
    iota = lax.iota(jnp.int32, L)

    def start_in(i, p):
        H0 = (start + i) * SB
        pltpu.async_copy(tt_hbm.at[pl.ds(0, D), pl.ds(H0, SB)],
                         bufs[p], semi[p])

    def transpose_buf(buf, obuf, nh):
        def hh_body(hg, c):
            h0 = hg * L
            base = (h0 + iota) * HW
            for k in range(HW):
                vE = buf[2 * k, pl.ds(h0, L)]
                vO = buf[2 * k + 1, pl.ds(h0, L)]
                pk = plsc.pack(vE, vO, format=plsc.PackFormat.INTERLEAVED)
                pw = plsc.bitcast(pk, jnp.int32)
                rot = jnp.bitwise_and(iota + k, L - 1)
                plsc.store_scatter(obuf, [base + rot], pw)
            return c

        lax.fori_loop(0, nh // L, hh_body, 0)

    for q in range(3):
        start_in(q, q)

    def outer(ib, c):
        for p in range(4):
            i = 4 * ib + p
            op = p % 2

            @pl.when(i < cnt)
            def _():
                @pl.when(i + 3 < cnt)
                def _():
                    start_in(i + 3, (p + 3) % 4)

                H0 = (start + i) * SB
                pltpu.make_async_copy(
                    tt_hbm.at[pl.ds(0, D), pl.ds(H0, SB)],
                    bufs[p], semi[p]).wait()

                @pl.when(i >= 2)
                def _():
                    pltpu.make_async_copy(
                        obufs[op], out_hbm.at[pl.ds(0, SB * HW)],
                        semo[op]).wait()

                transpose_buf(bufs[p], obufs[op], SB)
                pltpu.async_copy(obufs[op],
                                 out_hbm.at[pl.ds(H0 * HW, SB * HW)], semo[op])
        return c

    lax.fori_loop(0, SB_PER_W // 4 + 1, outer, 0)
    for p in range(2):
        pltpu.make_async_copy(obufs[p], out_hbm.at[pl.ds(0, SB * HW)],
                              semo[p]).wait()

    @pl.when(wid == NW - 1)
    def _():
        pltpu.sync_copy(tt_hbm.at[pl.ds(0, D), pl.ds(NSB * SB, TAIL_H)], tb)
        transpose_buf(tb, tob, TAIL_H)
        pltpu.sync_copy(tob, out_hbm.at[pl.ds(NSB * SB * HW, TAIL_H * HW)])


def _body(hashes_hbm, wts_hbm, table_hbm, out_hbm,
          idx0, wts0, rows0, idx1, wts1, rows1, outb_v, sem0, sem1):
    wid = lax.axis_index("s") * NC + lax.axis_index("c")
    idx = (idx0, idx1)
    wts = (wts0, wts1)
    rows = (rows0, rows1)
    sem = (sem0, sem1)
    iota = lax.iota(jnp.int32, L)
    oE_base = 2 * iota

    def stage(g, p):
        base_g = (wid * RW + g * CB) * HIST
        pltpu.sync_copy(hashes_hbm.at[pl.ds(base_g, GC)],
                        idx[p].at[pl.ds(0, GC)])
        pltpu.sync_copy(wts_hbm.at[pl.ds(base_g, GC)],
                        wts[p].at[pl.ds(0, GC)])

        def wm_body(j, c):
            iv = idx[p][pl.ds(j * L, L)]
            wv = wts[p][pl.ds(j * L, L)]
            wts[p][pl.ds(j * L, L)] = jnp.where(iv == PADDING_IDX, 0.0, wv)
            return c

        lax.fori_loop(0, GC // L, wm_body, 0)
        pltpu.async_copy(table_hbm.at[idx[p].at[pl.ds(0, GC)]],
                         rows[p], sem[p])

    def row_pair(rv, r, h):
        rot = jnp.bitwise_and(iota + h, L - 1)
        pw = plsc.load_gather(rv, [jnp.full((L,), r, jnp.int32), rot])
        pk = plsc.bitcast(pw, jnp.bfloat16)
        return plsc.unpack(pk, format=plsc.PackFormat.INTERLEAVED)

    def consume(g, p):
        pltpu.make_async_copy(table_hbm.at[idx[p].at[pl.ds(0, GC)]],
                              rows[p], sem[p]).wait()
        rv, wv, hv = rows[p], wts[p], idx[p]

        def row_body(b, c):
            r0 = b * HIST

            def k_body(k, acc):
                a0, a1, b0, b1 = acc
                rk = r0 + k * L
                w16 = wv[pl.ds(rk, L)]
                h16 = hv[pl.ds(rk, L)]
                for j in range(0, L, 2):
                    w = w16[j]
                    e, o = row_pair(rv, rk + j, h16[j])
                    a0 = a0 + w * e
                    a1 = a1 + w * o
                    w2 = w16[j + 1]
                    e2, o2 = row_pair(rv, rk + j + 1, h16[j + 1])
                    b0 = b0 + w2 * e2
                    b1 = b1 + w2 * o2
                return (a0, a1, b0, b1)

            z = jnp.zeros((L,), jnp.float32)
            a0, a1, b0, b1 = lax.fori_loop(0, HIST // L, k_body, (z, z, z, z))
            rt = r0 + (HIST // L) * L
            wt16 = wv[pl.ds(rt, L)]
            ht16 = hv[pl.ds(rt, L)]
            e, o = row_pair(rv, rt, ht16[0])
            a0 = a0 + wt16[0] * e
            a1 = a1 + wt16[0] * o
            e2, o2 = row_pair(rv, rt + 1, ht16[1])
            b0 = b0 + wt16[1] * e2
            b1 = b1 + wt16[1] * o2
            ob = b * D
            plsc.store_scatter(outb_v, [oE_base + ob], a0 + b0)
            plsc.store_scatter(outb_v, [oE_base + (ob + 1)], a1 + b1)
            return c

        lax.fori_loop(0, CB, row_body, 0)
        base_b = wid * RW + g * CB
        pltpu.sync_copy(outb_v, out_hbm.at[pl.ds(base_b * D, CB * D)])

    stage(0, 0)

    def outer(gb, c):
        for p in range(2):
            g = 2 * gb + p

            @pl.when(g + 1 < NCHUNK)
            def _():
                stage(g + 1, 1 - p)

            consume(g, p)
        return c

    lax.fori_loop(0, NCHUNK // 2, outer, 0)


@jax.jit
def kernel(hashes, weights, table):
    hashes_flat = hashes.astype(jnp.int32).reshape(B * HIST)
    weights_flat = weights.reshape(B * HIST)
    mesh = plsc.VectorSubcoreMesh(core_axis_name="c", subcore_axis_name="s")

    t_flat = pl.kernel(
        _tbody,
        out_type=jax.ShapeDtypeStruct((NUM_EMBEDDINGS * HW,), jnp.int32),
        mesh=mesh,
        scratch_types=[
            pltpu.VMEM((D, SB), jnp.float32),
            pltpu.VMEM((D, SB), jnp.float32),
            pltpu.VMEM((D, SB), jnp.float32),
            pltpu.VMEM((D, SB), jnp.float32),
            pltpu.VMEM((SB * HW,), jnp.int32),
            pltpu.VMEM((SB * HW,), jnp.int32),
            pltpu.SemaphoreType.DMA,
            pltpu.SemaphoreType.DMA,
            pltpu.SemaphoreType.DMA,
            pltpu.SemaphoreType.DMA,
            pltpu.SemaphoreType.DMA,
            pltpu.SemaphoreType.DMA,
            pltpu.VMEM((D, TAIL_H), jnp.float32),
            pltpu.VMEM((TAIL_H * HW,), jnp.int32),
        ],
        compiler_params=pltpu.CompilerParams(use_tc_tiling_on_sc=True,
                                             needs_layout_passes=False),
    )(table.T)
    t_pk = t_flat.reshape(NUM_EMBEDDINGS, HW)

    out_flat = pl.kernel(
        _body,
        out_type=jax.ShapeDtypeStruct((B * D,), jnp.float32),
        mesh=mesh,
        scratch_types=[
            pltpu.VMEM((GC + L,), jnp.int32),
            pltpu.VMEM((GC + L,), jnp.float32),
            pltpu.VMEM((GC, HW), jnp.int32),
            pltpu.VMEM((GC + L,), jnp.int32),
            pltpu.VMEM((GC + L,), jnp.float32),
            pltpu.VMEM((GC, HW), jnp.int32),
            pltpu.VMEM((CB * D,), jnp.float32),
            pltpu.SemaphoreType.DMA,
            pltpu.SemaphoreType.DMA,
        ],
        compiler_params=pltpu.CompilerParams(use_tc_tiling_on_sc=False,
                                             needs_layout_passes=False),
    )
    return out_flat(hashes_flat, weights_flat, t_pk).reshape(B, D)


# speedup vs baseline: 1.1777x; 1.1777x over previous
import jax
import jax.numpy as jnp
from jax import lax
from jax.experimental import pallas as pl
from jax.experimental.pallas import tpu as pltpu
from jax.experimental.pallas import tpu_sc as plsc

NUM_EMBEDDINGS = 1000000
D = 32
PADDING_IDX = 0
B = 16384
HIST = 50

L = 16
NC, NS = 2, 16
NW = NC * NS
RW = B // NW
CB = 32
GC = CB * HIST
NCHUNK = RW // CB

SB = 512
NSB = NUM_EMBEDDINGS // SB
TAIL_H = NUM_EMBEDDINGS - NSB * SB
SB_PER_W = 62


def _tbody(tt_hbm, out_hbm, b0, b1, o0, o1, si0, si1, so0, so1, tb, tob):
    wid = lax.axis_index("s") * NC + lax.axis_index("c")
    start = 61 * wid + jnp.minimum(wid, 1)
    cnt = jnp.where(wid < 1, 62, 61)
    bufs = (b0, b1)
    obufs = (o0, o1)
    semi = (si0, si1)
    semo = (so0, so1)
    iota = lax.iota(jnp.int32, L)

    def start_in(i, p):
        H0 = (start + i) * SB
        pltpu.async_copy(tt_hbm.at[pl.ds(0, D), pl.ds(H0, SB)],
                         bufs[p], semi[p])

    def transpose_buf(buf, obuf, nh):
        def hh_body(hg, c):
            h0 = hg * L
            h0b = h0 * D
            for r in range(L):
                t = jnp.bitwise_and(iota + r, L - 1)
                hh = h0 + t
                vA = plsc.load_gather(buf, [iota, hh])
                vB = plsc.load_gather(buf, [iota + L, hh])
                oA = t * D + iota + h0b
                plsc.store_scatter(obuf, [oA], vA)
                plsc.store_scatter(obuf, [oA + L], vB)
            return c

        lax.fori_loop(0, nh // L, hh_body, 0)

    start_in(0, 0)

    def outer(ib, c):
        for p in range(2):
            i = 2 * ib + p

            @pl.when(i < cnt)
            def _():
                @pl.when(i + 1 < cnt)
                def _():
                    start_in(i + 1, 1 - p)

                H0 = (start + i) * SB
                pltpu.make_async_copy(
                    tt_hbm.at[pl.ds(0, D), pl.ds(H0, SB)],
                    bufs[p], semi[p]).wait()

                @pl.when(i >= 2)
                def _():
                    pltpu.make_async_copy(
                        obufs[p], out_hbm.at[pl.ds(0, SB * D)],
                        semo[p]).wait()

                transpose_buf(bufs[p], obufs[p], SB)
                pltpu.async_copy(obufs[p],
                                 out_hbm.at[pl.ds(H0 * D, SB * D)], semo[p])
        return c

    lax.fori_loop(0, SB_PER_W // 2, outer, 0)
    for p in range(2):
        pltpu.make_async_copy(obufs[p], out_hbm.at[pl.ds(0, SB * D)],
                              semo[p]).wait()

    @pl.when(wid == NW - 1)
    def _():
        pltpu.sync_copy(tt_hbm.at[pl.ds(0, D), pl.ds(NSB * SB, TAIL_H)], tb)
        transpose_buf(tb, tob, TAIL_H)
        pltpu.sync_copy(tob, out_hbm.at[pl.ds(NSB * SB * D, TAIL_H * D)])


def _body(hashes_hbm, wts_hbm, table_hbm, out_hbm,
          idx0, wts0, idx1, wts1, idx2, wts2, rows0, rows1, outb_v,
          semi0, semi1, semi2, sem0, sem1):
    wid = lax.axis_index("s") * NC + lax.axis_index("c")
    idx = (idx0, idx1, idx2)
    wts = (wts0, wts1, wts2)
    semi = (semi0, semi1, semi2)
    rows = (rows0, rows1)
    sem = (sem0, sem1)

    def load_start(g, q):
        base_g = (wid * RW + g * CB) * HIST
        pltpu.async_copy(hashes_hbm.at[pl.ds(base_g, GC)], idx[q], semi[q])
        pltpu.async_copy(wts_hbm.at[pl.ds(base_g, GC)],
                         wts[q].at[pl.ds(0, GC)], semi[q])

    def gather_start(g, q, p):
        base_g = (wid * RW + g * CB) * HIST
        pltpu.make_async_copy(hashes_hbm.at[pl.ds(base_g, GC)], idx[q],
                              semi[q]).wait()
        pltpu.make_async_copy(wts_hbm.at[pl.ds(base_g, GC)],
                              wts[q].at[pl.ds(0, GC)], semi[q]).wait()

        def wm_body(j, c):
            iv = idx[q][pl.ds(j * L, L)]
            wv = wts[q][pl.ds(j * L, L)]
            wts[q][pl.ds(j * L, L)] = jnp.where(iv == PADDING_IDX, 0.0, wv)
            return c

        lax.fori_loop(0, GC // L, wm_body, 0)
        pltpu.async_copy(table_hbm.at[idx[q]], rows[p], sem[p])

    def consume(g, q, p):
        pltpu.make_async_copy(table_hbm.at[idx[q]], rows[p], sem[p]).wait()
        rv, wv = rows[p], wts[q]

        def row_body(b, c):
            r0 = b * HIST

            def k_body(k, acc):
                a0, a1, b0, b1 = acc
                rk = r0 + k * L
                w16 = wv[pl.ds(rk, L)]
                for j in range(0, L, 2):
                    w = w16[j]
                    a0 = a0 + w * rv[rk + j, pl.ds(0, L)]
                    a1 = a1 + w * rv[rk + j, pl.ds(L, L)]
                    w2 = w16[j + 1]
                    b0 = b0 + w2 * rv[rk + j + 1, pl.ds(0, L)]
                    b1 = b1 + w2 * rv[rk + j + 1, pl.ds(L, L)]
                return (a0, a1, b0, b1)

            z = jnp.zeros((L,), jnp.float32)
            a0, a1, b0, b1 = lax.fori_loop(0, HIST // L, k_body, (z, z, z, z))
            rt = r0 + (HIST // L) * L
            wt16 = wv[pl.ds(rt, L)]
            a0 = a0 + wt16[0] * rv[rt, pl.ds(0, L)]
            a1 = a1 + wt16[0] * rv[rt, pl.ds(L, L)]
            b0 = b0 + wt16[1] * rv[rt + 1, pl.ds(0, L)]
            b1 = b1 + wt16[1] * rv[rt + 1, pl.ds(L, L)]
            outb_v[b, pl.ds(0, L)] = a0 + b0
            outb_v[b, pl.ds(L, L)] = a1 + b1
            return c

        lax.fori_loop(0, CB, row_body, 0)
        base_b = wid * RW + g * CB
        pltpu.sync_copy(outb_v, out_hbm.at[pl.ds(base_b, CB)])

    load_start(0, 0)
    load_start(1, 1)
    gather_start(0, 0, 0)

    def outer(gb, c):
        for u in range(6):
            g = 6 * gb + u

            @pl.when(g + 2 < NCHUNK)
            def _():
                load_start(g + 2, (u + 2) % 3)

            @pl.when(g + 1 < NCHUNK)
            def _():
                gather_start(g + 1, (u + 1) % 3, (u + 1) % 2)

            @pl.when(g < NCHUNK)
            def _():
                consume(g, u % 3, u % 2)
        return c

    lax.fori_loop(0, (NCHUNK + 5) // 6, outer, 0)


@jax.jit
def kernel(hashes, weights, table):
    hashes_flat = hashes.astype(jnp.int32).reshape(B * HIST)
    weights_flat = weights.reshape(B * HIST)
    mesh = plsc.VectorSubcoreMesh(core_axis_name="c", subcore_axis_name="s")

    t_flat = pl.kernel(
        _tbody,
        out_type=jax.ShapeDtypeStruct((NUM_EMBEDDINGS * D,), jnp.float32),
        mesh=mesh,
        scratch_types=[
            pltpu.VMEM((D, SB), jnp.float32),
            pltpu.VMEM((D, SB), jnp.float32),
            pltpu.VMEM((SB * D,), jnp.float32),
            pltpu.VMEM((SB * D,), jnp.float32),
            pltpu.SemaphoreType.DMA,
            pltpu.SemaphoreType.DMA,
            pltpu.SemaphoreType.DMA,
            pltpu.SemaphoreType.DMA,
            pltpu.VMEM((D, TAIL_H), jnp.float32),
            pltpu.VMEM((TAIL_H * D,), jnp.float32),
        ],
        compiler_params=pltpu.CompilerParams(use_tc_tiling_on_sc=True,
                                             needs_layout_passes=False),
    )(table.T)
    t_rm = t_flat.reshape(NUM_EMBEDDINGS, D)

    run = pl.kernel(
        _body,
        out_type=jax.ShapeDtypeStruct((B, D), jnp.float32),
        mesh=mesh,
        scratch_types=[
            pltpu.VMEM((GC,), jnp.int32),
            pltpu.VMEM((GC + L,), jnp.float32),
            pltpu.VMEM((GC,), jnp.int32),
            pltpu.VMEM((GC + L,), jnp.float32),
            pltpu.VMEM((GC,), jnp.int32),
            pltpu.VMEM((GC + L,), jnp.float32),
            pltpu.VMEM((GC, D), jnp.float32),
            pltpu.VMEM((GC, D), jnp.float32),
            pltpu.VMEM((CB, D), jnp.float32),
            pltpu.SemaphoreType.DMA,
            pltpu.SemaphoreType.DMA,
            pltpu.SemaphoreType.DMA,
            pltpu.SemaphoreType.DMA,
            pltpu.SemaphoreType.DMA,
        ],
        compiler_params=pltpu.CompilerParams(use_tc_tiling_on_sc=False),
    )
    return run(hashes_flat, weights_flat, t_rm)
